# baseline (speedup 1.0000x reference)
import functools

import jax
import jax.numpy as jnp
from jax import lax
from jax.experimental import pallas as pl
from jax.experimental.pallas import tpu as pltpu
from jax.experimental.pallas import tpu_sc as plsc

B = 16384
D = 32
NC = 2
NS = 16
NW = NC * NS
BPW = B // NW
NCHUNK = 4
CHUNK = BPW // NCHUNK
GPC = CHUNK // 16

_CLAMP = 1.0 + 1e-7
_KEXP = 0.1353352832366127
_MAGIC = 0x5F3759DF


def _fermi_dirac_from_z(z):
    z = jnp.maximum(z, _CLAMP)
    w = (z - 1.0) * (z + 1.0)
    i = plsc.bitcast(w, jnp.int32)
    r = plsc.bitcast(_MAGIC - (i >> 1), jnp.float32)
    r = r * (1.5 - 0.5 * w * r * r)
    r = r * (1.5 - 0.5 * w * r * r)
    r = r * (1.5 - 0.5 * w * r * r)
    s = w * r
    return 1.0 / (_KEXP * (z + s) + 1.0)


def _make_kernel():
    mesh = plsc.VectorSubcoreMesh(core_axis_name="c", subcore_axis_name="s")

    @functools.partial(
        pl.kernel,
        out_type=jax.ShapeDtypeStruct((B,), jnp.float32),
        mesh=mesh,
        compiler_params=pltpu.CompilerParams(
            use_tc_tiling_on_sc=False, needs_layout_passes=False),
        scratch_types=[
            pltpu.VMEM((NCHUNK, CHUNK), jnp.int32),
            pltpu.VMEM((NCHUNK, CHUNK), jnp.int32),
            pltpu.VMEM((BPW, D), jnp.float32),
            pltpu.VMEM((BPW, D), jnp.float32),
            pltpu.VMEM((BPW,), jnp.float32),
            pltpu.SemaphoreType.DMA,
            pltpu.SemaphoreType.DMA,
            pltpu.SemaphoreType.DMA,
            pltpu.SemaphoreType.DMA,
        ],
    )
    def lorentz_fd(u_hbm, v_hbm, theta_hbm, out_hbm, ui, vi, ru, rv, ov,
                   s0, s1, s2, s3):
        sems = [s0, s1, s2, s3]
        wid = lax.axis_index("s") * NC + lax.axis_index("c")
        pltpu.sync_copy(u_hbm.at[wid], ui)
        pltpu.sync_copy(v_hbm.at[wid], vi)
        copies = []
        for j in range(NCHUNK):
            cu = pltpu.async_copy(theta_hbm.at[ui.at[j]],
                                  ru.at[pl.ds(j * CHUNK, CHUNK)], sems[j])
            cv = pltpu.async_copy(theta_hbm.at[vi.at[j]],
                                  rv.at[pl.ds(j * CHUNK, CHUNK)], sems[j])
            copies.append((cu, cv))

        iota16 = lax.iota(jnp.int32, 16)

        def group_body(g, carry):
            rid = g * 16 + iota16
            c0 = jnp.zeros((16,), jnp.int32)
            p0 = plsc.load_gather(ru, [rid, c0]) * plsc.load_gather(rv, [rid, c0])
            acc = jnp.zeros((16,), jnp.float32)
            for dd in range(1, D):
                cd = jnp.full((16,), dd, jnp.int32)
                acc = acc + (plsc.load_gather(ru, [rid, cd]) *
                             plsc.load_gather(rv, [rid, cd]))
            ov[pl.ds(g * 16, 16)] = _fermi_dirac_from_z(p0 - acc)
            return carry

        for j in range(NCHUNK):
            cu, cv = copies[j]
            cu.wait()
            cv.wait()
            lax.fori_loop(j * GPC, (j + 1) * GPC, group_body, 0)

        pltpu.sync_copy(ov, out_hbm.at[pl.ds(wid * BPW, BPW)])

    return lorentz_fd


_lorentz = _make_kernel()


def kernel(u, v, theta):
    u3 = u.astype(jnp.int32).reshape(NW, NCHUNK, CHUNK)
    v3 = v.astype(jnp.int32).reshape(NW, NCHUNK, CHUNK)
    return _lorentz(u3, v3, theta)
